# baseline (speedup 1.0000x reference)
import functools

import jax
import jax.numpy as jnp
from jax import lax
from jax.experimental import pallas as pl
from jax.experimental.pallas import tpu as pltpu
from jax.experimental.pallas import tpu_sc as plsc

B = 16
P = 4096
E = 16
H = 128
D = 128
L = 16
HP = P // 2
BIG = 1e19

CPN = 224


def _extract(cp_v, lanes, base, k):
    row = cp_v[pl.ds(base + (k // L) * L, L)]
    return jnp.sum(jnp.where(lanes == (k % L), row, jnp.float32(0.0)))


def _feature_body(pts0_hbm, pts1_hbm, cp_hbm, out_hbm, pv_v, cp_v, out_v):
    c = lax.axis_index("c")
    s = lax.axis_index("s")
    w = s * 2 + c
    pltpu.sync_copy(cp_hbm, cp_v)
    lanes = lax.iota(jnp.int32, L)
    zero = jnp.float32(0.0)

    for t in range(2):
        code = lax.convert_element_type(
            _extract(cp_v, lanes, 160, w + t * 32), jnp.int32)
        half = code % 2
        sfl = code // 2
        hc = sfl // B
        sb = sfl % B
        cnt = lax.convert_element_type(
            _extract(cp_v, lanes, 0, sfl), jnp.int32)
        start = half * HP
        npts = jnp.clip(cnt - start, 0, HP)
        nchunks = (npts + (L - 1)) // L

        @pl.when(hc == 0)
        def _():
            pltpu.sync_copy(pts0_hbm.at[sb, pl.ds(start, HP)], pv_v.at[pl.ds(0, HP)])
            pltpu.sync_copy(pts0_hbm.at[sb, pl.ds(P + start, HP)], pv_v.at[pl.ds(HP, HP)])

        @pl.when(hc == 1)
        def _():
            pltpu.sync_copy(pts1_hbm.at[sb, pl.ds(start, HP)], pv_v.at[pl.ds(0, HP)])
            pltpu.sync_copy(pts1_hbm.at[sb, pl.ds(P + start, HP)], pv_v.at[pl.ds(HP, HP)])

        pbase = 32 + hc * 64
        cx_row = cp_v[pl.ds(pbase, L)]
        cy_row = cp_v[pl.ds(pbase + L, L)]
        nsx_row = cp_v[pl.ds(pbase + 2 * L, L)]
        nsy_row = cp_v[pl.ds(pbase + 3 * L, L)]
        cxe, cye, nsxe, nsye = [], [], [], []
        for e in range(E):
            sel = lanes == e
            cxe.append(jnp.sum(jnp.where(sel, cx_row, zero)))
            cye.append(jnp.sum(jnp.where(sel, cy_row, zero)))
            nsxe.append(jnp.sum(jnp.where(sel, nsx_row, zero)))
            nsye.append(jnp.sum(jnp.where(sel, nsy_row, zero)))

        @pl.when(nchunks > 0)
        def _():
            i = nchunks - 1
            valid = (lanes + i * L) < npts
            pv_v[pl.ds(i * L, L)] = jnp.where(
                valid, pv_v[pl.ds(i * L, L)], jnp.full((L,), BIG, jnp.float32))
            pv_v[pl.ds(HP + i * L, L)] = jnp.where(
                valid, pv_v[pl.ds(HP + i * L, L)], jnp.zeros((L,), jnp.float32))

        G = 4
        out = jnp.zeros((L,), jnp.float32)
        for g in range(0, E, G):
            def group_chunk(i, accs):
                xv = pv_v[pl.ds(i * L, L)]
                yv = pv_v[pl.ds(HP + i * L, L)]
                new = []
                for j in range(G):
                    e = g + j
                    dx = xv - cxe[e]
                    dy = yv - cye[e]
                    tt = nsxe[e] * (dx * dx) + nsye[e] * (dy * dy)
                    tt = jnp.maximum(tt, -20000.0)
                    new.append(accs[j] + jnp.exp(tt))
                return tuple(new)

            accs = lax.fori_loop(0, nchunks, group_chunk,
                                 tuple(jnp.zeros((L,), jnp.float32) for _ in range(G)))
            for j in range(G):
                out = out + jnp.where(lanes == (g + j), jnp.sum(accs[j]), zero)

        out_v[...] = out
        pltpu.sync_copy(out_v, out_hbm.at[half, sb // 8, sb % 8, pl.ds(hc * E, E)])


def _mlp_body(f_ref, w1_ref, w2_ref, g1_ref, b1_ref, g2_ref, b2_ref, o_ref):
    x = f_ref[0, :, :2 * E] + f_ref[1, :, :2 * E]
    dn = (((1,), (1,)), ((), ()))
    hdn = lax.dot_general(x, w1_ref[...], dn, preferred_element_type=jnp.float32)
    mean = jnp.mean(hdn, axis=0, keepdims=True)
    var = jnp.mean((hdn - mean) * (hdn - mean), axis=0, keepdims=True)
    hdn = (hdn - mean) / jnp.sqrt(var + 1e-5) * g1_ref[...][None, :] + b1_ref[...][None, :]
    hdn = jnp.maximum(hdn, 0.0)
    y = lax.dot_general(hdn, w2_ref[...], dn, preferred_element_type=jnp.float32)
    mean2 = jnp.mean(y, axis=0, keepdims=True)
    var2 = jnp.mean((y - mean2) * (y - mean2), axis=0, keepdims=True)
    y = (y - mean2) / jnp.sqrt(var2 + 1e-5) * g2_ref[...][None, :] + b2_ref[...][None, :]
    nrm = jnp.maximum(jnp.sqrt(jnp.sum(y * y, axis=1, keepdims=True)), 1e-12)
    o_ref[...] = y / nrm


@functools.partial(
    pl.kernel,
    out_type=jax.ShapeDtypeStruct((2, 2, 8, 128), jnp.float32),
    mesh=plsc.VectorSubcoreMesh(core_axis_name="c", subcore_axis_name="s"),
    compiler_params=pltpu.CompilerParams(needs_layout_passes=False),
    scratch_types=[
        pltpu.VMEM((P,), jnp.float32),
        pltpu.VMEM((CPN,), jnp.float32),
        pltpu.VMEM((L,), jnp.float32),
    ],
)
def _features(*refs):
    _feature_body(*refs)


_mlp = pl.pallas_call(
    _mlp_body,
    out_shape=jax.ShapeDtypeStruct((B, D), jnp.float32),
)


def kernel(barcode_h0, barcode_h0_count, barcode_h1, barcode_h1_count,
           centers_h0, log_sharpness_h0, centers_h1, log_sharpness_h1,
           W1, W2, bn1_gamma, bn1_beta, bn2_gamma, bn2_beta):
    pts0 = jnp.transpose(barcode_h0, (0, 2, 1)).reshape(B, 2 * P)
    pts1 = jnp.transpose(barcode_h1, (0, 2, 1)).reshape(B, 2 * P)
    nsharp0 = -(jax.nn.softplus(log_sharpness_h0) + 1e-6)
    nsharp1 = -(jax.nn.softplus(log_sharpness_h1) + 1e-6)
    counts_all = jnp.concatenate([barcode_h0_count, barcode_h1_count]).astype(jnp.int32)
    sizes = jnp.stack([jnp.minimum(counts_all, HP),
                       jnp.maximum(counts_all - HP, 0)], axis=1).reshape(64)
    order = jnp.argsort(-sizes).astype(jnp.int32)
    table = jnp.concatenate([order[:32], order[32:][::-1]])
    cp = jnp.concatenate([
        counts_all.astype(jnp.float32),
        centers_h0[:, 0], centers_h0[:, 1], nsharp0[:, 0], nsharp0[:, 1],
        centers_h1[:, 0], centers_h1[:, 1], nsharp1[:, 0], nsharp1[:, 1],
        table.astype(jnp.float32),
    ])
    f = _features(pts0, pts1, cp).reshape(2, B, D)
    return _mlp(f, W1, W2, bn1_gamma, bn1_beta, bn2_gamma, bn2_beta)


# speedup vs baseline: 1.0823x; 1.0823x over previous
import functools

import jax
import jax.numpy as jnp
from jax import lax
from jax.experimental import pallas as pl
from jax.experimental.pallas import tpu as pltpu
from jax.experimental.pallas import tpu_sc as plsc

B = 16
P = 4096
E = 16
H = 128
D = 128
L = 16
NCH = P // L
BIG = 1e19


def _feature_body(pts0_hbm, pts1_hbm, cp_hbm, out_hbm,
                  pv_v, cp_v, out_v):
    h = lax.axis_index("c")
    b = lax.axis_index("s")
    pltpu.sync_copy(cp_hbm.at[h], cp_v)

    @pl.when(h == 0)
    def _():
        pltpu.sync_copy(pts0_hbm.at[b], pv_v)

    @pl.when(h == 1)
    def _():
        pltpu.sync_copy(pts1_hbm.at[b], pv_v)

    lanes = lax.iota(jnp.int32, L)
    zero = jnp.float32(0.0)
    cnt_row = cp_v[pl.ds(0, L)]
    cnt = lax.convert_element_type(
        jnp.sum(jnp.where(lanes == b, cnt_row, zero)), jnp.int32)
    nchunks = (cnt + (L - 1)) // L

    cx_row = cp_v[pl.ds(L, L)]
    cy_row = cp_v[pl.ds(2 * L, L)]
    nsx_row = cp_v[pl.ds(3 * L, L)]
    nsy_row = cp_v[pl.ds(4 * L, L)]
    cxe, cye, nsxe, nsye = [], [], [], []
    for e in range(E):
        sel = lanes == e
        cxe.append(jnp.sum(jnp.where(sel, cx_row, zero)))
        cye.append(jnp.sum(jnp.where(sel, cy_row, zero)))
        nsxe.append(jnp.sum(jnp.where(sel, nsx_row, zero)))
        nsye.append(jnp.sum(jnp.where(sel, nsy_row, zero)))

    @pl.when(nchunks > 0)
    def _():
        i = nchunks - 1
        valid = (lanes + i * L) < cnt
        pv_v[pl.ds(i * L, L)] = jnp.where(
            valid, pv_v[pl.ds(i * L, L)], jnp.full((L,), BIG, jnp.float32))
        pv_v[pl.ds(P + i * L, L)] = jnp.where(
            valid, pv_v[pl.ds(P + i * L, L)], jnp.zeros((L,), jnp.float32))

    G = 8
    out = jnp.zeros((L,), jnp.float32)
    for g in range(0, E, G):
        def group_chunk(i, accs):
            xv = pv_v[pl.ds(i * L, L)]
            yv = pv_v[pl.ds(P + i * L, L)]
            new = []
            for j in range(G):
                e = g + j
                dx = xv - cxe[e]
                dy = yv - cye[e]
                t = nsxe[e] * (dx * dx) + nsye[e] * (dy * dy)
                new.append(accs[j] + jnp.exp(t))
            return tuple(new)

        accs = lax.fori_loop(0, nchunks, group_chunk,
                             tuple(jnp.zeros((L,), jnp.float32) for _ in range(G)))
        for j in range(G):
            out = out + jnp.where(lanes == (g + j), jnp.sum(accs[j]), zero)

    out_v[...] = out
    pltpu.sync_copy(out_v, out_hbm.at[b // 8, b % 8, pl.ds(h * E, E)])


def _mlp_body(f_ref, w1_ref, w2_ref, g1_ref, b1_ref, g2_ref, b2_ref, o_ref):
    x = f_ref[:, :2 * E]
    dn = (((1,), (1,)), ((), ()))
    hdn = lax.dot_general(x, w1_ref[...], dn, preferred_element_type=jnp.float32)
    mean = jnp.mean(hdn, axis=0, keepdims=True)
    var = jnp.mean((hdn - mean) * (hdn - mean), axis=0, keepdims=True)
    hdn = (hdn - mean) / jnp.sqrt(var + 1e-5) * g1_ref[...][None, :] + b1_ref[...][None, :]
    hdn = jnp.maximum(hdn, 0.0)
    y = lax.dot_general(hdn, w2_ref[...], dn, preferred_element_type=jnp.float32)
    mean2 = jnp.mean(y, axis=0, keepdims=True)
    var2 = jnp.mean((y - mean2) * (y - mean2), axis=0, keepdims=True)
    y = (y - mean2) / jnp.sqrt(var2 + 1e-5) * g2_ref[...][None, :] + b2_ref[...][None, :]
    nrm = jnp.maximum(jnp.sqrt(jnp.sum(y * y, axis=1, keepdims=True)), 1e-12)
    o_ref[...] = y / nrm


@functools.partial(
    pl.kernel,
    out_type=jax.ShapeDtypeStruct((2, 8, 128), jnp.float32),
    mesh=plsc.VectorSubcoreMesh(core_axis_name="c", subcore_axis_name="s"),
    compiler_params=pltpu.CompilerParams(needs_layout_passes=False),
    scratch_types=[
        pltpu.VMEM((2 * P,), jnp.float32),
        pltpu.VMEM((5 * L,), jnp.float32),
        pltpu.VMEM((L,), jnp.float32),
    ],
)
def _features(*refs):
    _feature_body(*refs)


_mlp = pl.pallas_call(
    _mlp_body,
    out_shape=jax.ShapeDtypeStruct((B, D), jnp.float32),
)


def kernel(barcode_h0, barcode_h0_count, barcode_h1, barcode_h1_count,
           centers_h0, log_sharpness_h0, centers_h1, log_sharpness_h1,
           W1, W2, bn1_gamma, bn1_beta, bn2_gamma, bn2_beta):
    pts0 = jnp.transpose(barcode_h0, (0, 2, 1)).reshape(B, 2 * P)
    pts1 = jnp.transpose(barcode_h1, (0, 2, 1)).reshape(B, 2 * P)
    nsharp0 = -(jax.nn.softplus(log_sharpness_h0) + 1e-6)
    nsharp1 = -(jax.nn.softplus(log_sharpness_h1) + 1e-6)
    cp = jnp.stack([
        jnp.concatenate([barcode_h0_count.astype(jnp.float32),
                         centers_h0[:, 0], centers_h0[:, 1],
                         nsharp0[:, 0], nsharp0[:, 1]]),
        jnp.concatenate([barcode_h1_count.astype(jnp.float32),
                         centers_h1[:, 0], centers_h1[:, 1],
                         nsharp1[:, 0], nsharp1[:, 1]]),
    ])
    f = _features(pts0, pts1, cp).reshape(B, D)
    return _mlp(f, W1, W2, bn1_gamma, bn1_beta, bn2_gamma, bn2_beta)
